# Initial kernel scaffold; baseline (speedup 1.0000x reference)
import jax
import jax.numpy as jnp
from jax.experimental import pallas as pl


def kernel(x, edge_index, edge_weight, W1, b1, W2, b2, W3, b3):
    raise NotImplementedError("write your pallas kernel here")


# speedup vs baseline: 7.3896x; 7.3896x over previous
import functools

import jax
import jax.numpy as jnp
from jax import lax
from jax.experimental import pallas as pl
from jax.experimental.pallas import tpu as pltpu
from jax.experimental.pallas import tpu_sc as plsc

N = 10000
NPAD = 10240
E = 320000
EPAD = 323584
CHUNK = 128
NSUB = 16
BN = 256
NB = NPAD // BN


def _mesh():
    return plsc.VectorSubcoreMesh(core_axis_name="c", subcore_axis_name="s")


def _fill_rows(ref, nrows, ncols, value):
    vals = jnp.full((16,), value, dtype=ref.dtype)

    def body(i, _):
        for j in range(ncols // 16):
            ref[i, pl.ds(j * 16, 16)] = vals
        return 0

    lax.fori_loop(0, nrows, body, 0)


def _hist_body(dst_hbm, out_hbm, dst_v, ones_v, stage_v, acc, sem):
    c = lax.axis_index("c")
    s = lax.axis_index("s")
    rows_per_tile = NPAD // NSUB
    row0 = s * rows_per_tile
    stage_rows = rows_per_tile // 4

    _fill_rows(stage_v, stage_rows, 128, 0.0)
    for k in range(4):
        pltpu.sync_copy(stage_v, acc.at[pl.ds(row0 + k * stage_rows, stage_rows)])
    _fill_rows(ones_v, CHUNK, 128, 1.0)
    plsc.subcore_barrier()

    per_core = EPAD // 2
    per_tile = per_core // NSUB
    nchunks = per_tile // CHUNK
    base0 = c * per_core + s * per_tile

    def body(i, _):
        base = base0 + i * CHUNK
        pltpu.sync_copy(dst_hbm.at[pl.ds(base, CHUNK)], dst_v)
        pltpu.sync_copy(ones_v, acc.at[dst_v], add=True)
        return 0

    lax.fori_loop(0, nchunks, body, 0)
    plsc.subcore_barrier()

    for k in range(4):
        pltpu.sync_copy(acc.at[pl.ds(row0 + k * stage_rows, stage_rows)], stage_v)
        pltpu.sync_copy(stage_v,
                        out_hbm.at[pl.ds(c * NPAD + row0 + k * stage_rows, stage_rows)])


def _make_hist():
    return pl.kernel(
        _hist_body,
        out_type=jax.ShapeDtypeStruct((2 * NPAD, 128), jnp.float32),
        mesh=_mesh(),
        scratch_types=[
            pltpu.VMEM((CHUNK,), jnp.int32),
            pltpu.VMEM((CHUNK, 128), jnp.float32),
            pltpu.VMEM((NPAD // NSUB // 4, 128), jnp.float32),
            pltpu.VMEM_SHARED((NPAD, 128), jnp.float32),
            pltpu.SemaphoreType.DMA,
        ],
    )


def _agg_split_body(g_hbm, src_hbm, dst_hbm, out_hbm,
                    src_v, src2_v, dst_v, rows_v, stage_v, acc, sem):
    c = lax.axis_index("c")
    s = lax.axis_index("s")
    rows_per_tile = NPAD // NSUB
    row0 = s * rows_per_tile
    stage_rows = rows_per_tile // 4

    _fill_rows(stage_v, stage_rows, 128, 0.0)
    for k in range(4):
        pltpu.sync_copy(stage_v, acc.at[pl.ds(row0 + k * stage_rows, stage_rows)])
    plsc.subcore_barrier()

    per_tile = EPAD // NSUB
    nchunks = per_tile // CHUNK
    base0 = s * per_tile
    off = c * NPAD

    def body(i, _):
        base = base0 + i * CHUNK
        pltpu.sync_copy(src_hbm.at[pl.ds(base, CHUNK)], src_v)
        pltpu.sync_copy(dst_hbm.at[pl.ds(base, CHUNK)], dst_v)
        for j in range(CHUNK // 16):
            src2_v[pl.ds(j * 16, 16)] = src_v[pl.ds(j * 16, 16)] + off
        pltpu.async_copy(g_hbm.at[src2_v], rows_v, sem).wait()
        pltpu.sync_copy(rows_v, acc.at[dst_v], add=True)
        return 0

    lax.fori_loop(0, nchunks, body, 0)
    plsc.subcore_barrier()

    for k in range(4):
        pltpu.sync_copy(acc.at[pl.ds(row0 + k * stage_rows, stage_rows)], stage_v)
        pltpu.sync_copy(stage_v,
                        out_hbm.at[pl.ds(c * NPAD + row0 + k * stage_rows, stage_rows)])


def _make_agg_split():
    return pl.kernel(
        _agg_split_body,
        out_type=jax.ShapeDtypeStruct((2 * NPAD, 128), jnp.float32),
        mesh=_mesh(),
        scratch_types=[
            pltpu.VMEM((CHUNK,), jnp.int32),
            pltpu.VMEM((CHUNK,), jnp.int32),
            pltpu.VMEM((CHUNK,), jnp.int32),
            pltpu.VMEM((CHUNK, 128), jnp.float32),
            pltpu.VMEM((NPAD // NSUB // 4, 128), jnp.float32),
            pltpu.VMEM_SHARED((NPAD, 128), jnp.float32),
            pltpu.SemaphoreType.DMA,
        ],
    )


def _agg64_body(g_hbm, src_hbm, dst_hbm, out_hbm,
                src_v, dst_v, rows_v, stage_v, acc, sem):
    c = lax.axis_index("c")
    s = lax.axis_index("s")
    rows_per_tile = NPAD // NSUB
    row0 = s * rows_per_tile
    stage_rows = rows_per_tile // 4

    _fill_rows(stage_v, stage_rows, 128, 0.0)
    for k in range(4):
        pltpu.sync_copy(stage_v, acc.at[pl.ds(row0 + k * stage_rows, stage_rows)])
    plsc.subcore_barrier()

    per_core = EPAD // 2
    per_tile = per_core // NSUB
    nchunks = per_tile // CHUNK
    base0 = c * per_core + s * per_tile

    def body(i, _):
        base = base0 + i * CHUNK
        pltpu.sync_copy(src_hbm.at[pl.ds(base, CHUNK)], src_v)
        pltpu.sync_copy(dst_hbm.at[pl.ds(base, CHUNK)], dst_v)
        pltpu.async_copy(g_hbm.at[src_v], rows_v, sem).wait()
        pltpu.sync_copy(rows_v, acc.at[dst_v], add=True)
        return 0

    lax.fori_loop(0, nchunks, body, 0)
    plsc.subcore_barrier()

    for k in range(4):
        pltpu.sync_copy(acc.at[pl.ds(row0 + k * stage_rows, stage_rows)], stage_v)
        pltpu.sync_copy(stage_v,
                        out_hbm.at[pl.ds(c * NPAD + row0 + k * stage_rows, stage_rows)])


def _make_agg64():
    return pl.kernel(
        _agg64_body,
        out_type=jax.ShapeDtypeStruct((2 * NPAD, 128), jnp.float32),
        mesh=_mesh(),
        scratch_types=[
            pltpu.VMEM((CHUNK,), jnp.int32),
            pltpu.VMEM((CHUNK,), jnp.int32),
            pltpu.VMEM((CHUNK, 128), jnp.float32),
            pltpu.VMEM((NPAD // NSUB // 4, 128), jnp.float32),
            pltpu.VMEM_SHARED((NPAD, 128), jnp.float32),
            pltpu.SemaphoreType.DMA,
        ],
    )


def _tc_l1_body(x_ref, w_ref, ca_ref, cb_ref, g_ref, d_ref):
    d = lax.rsqrt(ca_ref[...] + cb_ref[...] + 1.0)
    h = jnp.dot(x_ref[...], w_ref[...], preferred_element_type=jnp.float32)
    g_ref[...] = h * d
    d_ref[...] = d


def _tc_mid_body(sa_ref, sb_ref, ga_ref, gb_ref, d_ref, b_ref, w_ref,
                 out_ref):
    d = d_ref[...]
    b = b_ref[...]
    xa = jnp.maximum(d * (sa_ref[...] + ga_ref[...]) + b[:, :128], 0.0)
    xb = jnp.maximum(d * (sb_ref[...] + gb_ref[...]) + b[:, 128:], 0.0)
    x = jnp.concatenate([xa, xb], axis=1)
    h = jnp.dot(x, w_ref[...], preferred_element_type=jnp.float32)
    out_ref[...] = h * d


def _tc_out_body(sa_ref, sb_ref, g_ref, d_ref, b_ref, out_ref):
    out_ref[...] = d_ref[...] * (sa_ref[...] + sb_ref[...] + g_ref[...]) + b_ref[...]


@jax.jit
def _run(x, src, dst, W1, b1, W2, b2, W3, b3):
    srcp = jnp.concatenate([src, jnp.zeros((EPAD - E,), jnp.int32)])
    dstp = jnp.concatenate([dst, jnp.full((EPAD - E,), N, jnp.int32)])
    xp = jnp.pad(x, ((0, NPAD - N), (0, 0)))
    W3p = jnp.pad(W3, ((0, 0), (0, 128 - 40)))
    b1r = b1.reshape(1, 256)
    b2r = b2.reshape(1, 256)
    b3r = jnp.pad(b3, (0, 128 - 40)).reshape(1, 128)

    cnt = _make_hist()(dstp)
    cA = cnt[:NPAD]
    cB = cnt[NPAD:]

    rowA = pl.BlockSpec((BN, 128), lambda j, i: (i, 0))
    rowB = pl.BlockSpec((BN, 128), lambda j, i: (NB + i, 0))
    out_split = pl.BlockSpec((BN, 128), lambda j, i: (j * NB + i, 0))

    g1, dinv = pl.pallas_call(
        _tc_l1_body,
        grid=(2, NB),
        in_specs=[
            pl.BlockSpec((BN, 128), lambda j, i: (i, 0)),
            pl.BlockSpec((128, 128), lambda j, i: (0, j)),
            rowA,
            rowA,
        ],
        out_specs=[out_split, rowA],
        out_shape=[jax.ShapeDtypeStruct((2 * NPAD, 128), jnp.float32),
                   jax.ShapeDtypeStruct((NPAD, 128), jnp.float32)],
    )(xp, W1, cA, cB)

    s1 = _make_agg_split()(g1, srcp, dstp)

    g2 = pl.pallas_call(
        _tc_mid_body,
        grid=(2, NB),
        in_specs=[
            rowA, rowB, rowA, rowB, rowA,
            pl.BlockSpec((1, 256), lambda j, i: (0, 0)),
            pl.BlockSpec((256, 128), lambda j, i: (0, j)),
        ],
        out_specs=out_split,
        out_shape=jax.ShapeDtypeStruct((2 * NPAD, 128), jnp.float32),
    )(s1, s1, g1, g1, dinv, b1r, W2)

    s2 = _make_agg_split()(g2, srcp, dstp)

    rowA1 = pl.BlockSpec((BN, 128), lambda i: (i, 0))
    rowB1 = pl.BlockSpec((BN, 128), lambda i: (NB + i, 0))

    g3 = pl.pallas_call(
        _tc_mid_body,
        grid=(NB,),
        in_specs=[
            rowA1, rowB1, rowA1, rowB1, rowA1,
            pl.BlockSpec((1, 256), lambda i: (0, 0)),
            pl.BlockSpec((256, 128), lambda i: (0, 0)),
        ],
        out_specs=pl.BlockSpec((BN, 128), lambda i: (i, 0)),
        out_shape=jax.ShapeDtypeStruct((NPAD, 128), jnp.float32),
    )(s2, s2, g2, g2, dinv, b2r, W3p)

    s3 = _make_agg64()(g3, srcp, dstp)

    out = pl.pallas_call(
        _tc_out_body,
        grid=(NB,),
        in_specs=[
            rowA1, rowB1, rowA1, rowA1,
            pl.BlockSpec((1, 128), lambda i: (0, 0)),
        ],
        out_specs=pl.BlockSpec((BN, 128), lambda i: (i, 0)),
        out_shape=jax.ShapeDtypeStruct((NPAD, 128), jnp.float32),
    )(s3, s3, g3, dinv, b3r)

    return out[:N, :40]


def kernel(x, edge_index, edge_weight, W1, b1, W2, b2, W3, b3):
    del edge_weight
    return _run(x, edge_index[0], edge_index[1], W1, b1, W2, b2, W3, b3)
